# Initial kernel scaffold; baseline (speedup 1.0000x reference)
import functools

import jax
import jax.numpy as jnp
from jax import lax
from jax.experimental import pallas as pl
from jax.experimental.pallas import tpu as pltpu
from jax.experimental.pallas import tpu_sc as plsc

_N = 10000
_E = 320000
_H = 128
_G = 64
_C = 10

_B = 80
_EROWS = _E // _B
_NW = 32
_RPW = _EROWS // _NW
_D = 5
_NG = _RPW // _D
_TROWS = _N // 16
_BUFR = _D * _B

_TCB = 1000
_TCG = _N // _TCB


def _sc_mesh():
    return plsc.VectorSubcoreMesh(core_axis_name="c", subcore_axis_name="s")


def _deg_sc(dst2):
    @functools.partial(
        pl.kernel,
        out_type=jax.ShapeDtypeStruct((2 * _N, 16), jnp.float32),
        mesh=_sc_mesh(),
        scratch_types=[
            pltpu.VMEM((_RPW, _B), jnp.int32),
            pltpu.VMEM((_B, 16), jnp.float32),
            pltpu.VMEM((_B, 16), jnp.float32),
            pltpu.VMEM_SHARED((_N, 16), jnp.float32),
        ],
    )
    def deg_kernel(dst_hbm, out_hbm, didx, ones_v, zbuf, hist):
        cid = lax.axis_index("c")
        sid = lax.axis_index("s")
        wid = cid * 16 + sid
        tid = sid

        def fill(r, _):
            ones_v[r, :] = jnp.full((16,), 1.0, jnp.float32)
            zbuf[r, :] = jnp.zeros((16,), jnp.float32)
            return 0

        lax.fori_loop(0, _B, fill, 0)

        base = tid * _TROWS
        for k in range(_TROWS // _B):
            pltpu.sync_copy(zbuf, hist.at[pl.ds(base + k * _B, _B)])
        rem = _TROWS - (_TROWS // _B) * _B
        if rem:
            pltpu.sync_copy(zbuf.at[pl.ds(0, rem)],
                            hist.at[pl.ds(base + (_TROWS // _B) * _B, rem)])
        plsc.subcore_barrier()

        pltpu.sync_copy(dst_hbm.at[pl.ds(wid * _RPW, _RPW)], didx)

        def body(r, _):
            pltpu.sync_copy(ones_v, hist.at[didx.at[r]], add=True)
            return 0

        lax.fori_loop(0, _RPW, body, 0)
        plsc.subcore_barrier()

        obase = cid * _N + base
        for k in range(_TROWS // _B):
            pltpu.sync_copy(hist.at[pl.ds(base + k * _B, _B)], zbuf)
            pltpu.sync_copy(zbuf, out_hbm.at[pl.ds(obase + k * _B, _B)])
        if rem:
            pltpu.sync_copy(hist.at[pl.ds(base + (_TROWS // _B) * _B, rem)],
                            zbuf.at[pl.ds(0, rem)])
            pltpu.sync_copy(zbuf.at[pl.ds(0, rem)],
                            out_hbm.at[pl.ds(obase + (_TROWS // _B) * _B, rem)])

    return deg_kernel(dst2)


def _agg_sc(xw, src2, dst2):
    @functools.partial(
        pl.kernel,
        out_type=jax.ShapeDtypeStruct((2 * _N, _H), jnp.float32),
        mesh=_sc_mesh(),
        scratch_types=[
            pltpu.VMEM((_RPW, _B), jnp.int32),
            pltpu.VMEM((_RPW, _B), jnp.int32),
            pltpu.VMEM((_BUFR, _H), jnp.float32),
            pltpu.VMEM_SHARED((_N, _H), jnp.float32),
            pltpu.SemaphoreType.DMA,
        ],
    )
    def agg_kernel(xw_hbm, src_hbm, dst_hbm, out_hbm, sidx, didx, buf, acc, sem):
        cid = lax.axis_index("c")
        sid = lax.axis_index("s")
        wid = cid * 16 + sid
        tid = sid

        def zero(r, _):
            for j in range(_H // 16):
                buf[r, pl.ds(j * 16, 16)] = jnp.zeros((16,), jnp.float32)
            return 0

        lax.fori_loop(0, _BUFR, zero, 0)

        base = tid * _TROWS
        pltpu.sync_copy(buf, acc.at[pl.ds(base, _BUFR)])
        pltpu.sync_copy(buf.at[pl.ds(0, _TROWS - _BUFR)],
                        acc.at[pl.ds(base + _BUFR, _TROWS - _BUFR)])
        plsc.subcore_barrier()

        pltpu.sync_copy(src_hbm.at[pl.ds(wid * _RPW, _RPW)], sidx)
        pltpu.sync_copy(dst_hbm.at[pl.ds(wid * _RPW, _RPW)], didx)

        def body(g, _):
            descs = []
            for d in range(_D):
                descs.append(pltpu.async_copy(
                    xw_hbm.at[sidx.at[g * _D + d]],
                    buf.at[pl.ds(d * _B, _B)], sem))
            for d in range(_D):
                descs[d].wait()
            for d in range(_D):
                pltpu.sync_copy(buf.at[pl.ds(d * _B, _B)],
                                acc.at[didx.at[g * _D + d]], add=True)
            return 0

        lax.fori_loop(0, _NG, body, 0)
        plsc.subcore_barrier()

        obase = cid * _N + base
        pltpu.sync_copy(acc.at[pl.ds(base, _BUFR)], buf)
        pltpu.sync_copy(buf, out_hbm.at[pl.ds(obase, _BUFR)])
        rem = _TROWS - _BUFR
        pltpu.sync_copy(acc.at[pl.ds(base + _BUFR, rem)], buf.at[pl.ds(0, rem)])
        pltpu.sync_copy(buf.at[pl.ds(0, rem)], out_hbm.at[pl.ds(obase + _BUFR, rem)])

    return agg_kernel(xw, src2, dst2)


def _dinv_from_hist(hA, hB):
    deg = hA[:, 0:1] + hB[:, 0:1] + 1.0
    return lax.rsqrt(deg)


def _prep_body(x_ref, w_ref, hA_ref, hB_ref, o_ref):
    dinv = _dinv_from_hist(hA_ref[...], hB_ref[...])
    xw = jnp.dot(x_ref[...], w_ref[...], preferred_element_type=jnp.float32)
    o_ref[...] = dinv * xw


def _prep_tc(x, W, hist):
    return pl.pallas_call(
        _prep_body,
        grid=(_TCG,),
        in_specs=[
            pl.BlockSpec((_TCB, _H), lambda i: (i, 0)),
            pl.BlockSpec((_H, _H), lambda i: (0, 0)),
            pl.BlockSpec((_TCB, 16), lambda i: (i, 0)),
            pl.BlockSpec((_TCB, 16), lambda i: (i + _TCG, 0)),
        ],
        out_specs=pl.BlockSpec((_TCB, _H), lambda i: (i, 0)),
        out_shape=jax.ShapeDtypeStruct((_N, _H), jnp.float32),
    )(x, W, hist, hist)


def _mid_body(a0_ref, a1_ref, xw_ref, b_ref, w_ref, hA_ref, hB_ref, o_ref):
    dinv = _dinv_from_hist(hA_ref[...], hB_ref[...])
    pre = dinv * (a0_ref[...] + a1_ref[...] + xw_ref[...]) + b_ref[...]
    h = jnp.maximum(pre, 0.0)
    o_ref[...] = dinv * jnp.dot(h, w_ref[...], preferred_element_type=jnp.float32)


def _mid_tc(agg, xw, b, W2, hist):
    return pl.pallas_call(
        _mid_body,
        grid=(_TCG,),
        in_specs=[
            pl.BlockSpec((_TCB, _H), lambda i: (i, 0)),
            pl.BlockSpec((_TCB, _H), lambda i: (i + _TCG, 0)),
            pl.BlockSpec((_TCB, _H), lambda i: (i, 0)),
            pl.BlockSpec((1, _H), lambda i: (0, 0)),
            pl.BlockSpec((_H, _H), lambda i: (0, 0)),
            pl.BlockSpec((_TCB, 16), lambda i: (i, 0)),
            pl.BlockSpec((_TCB, 16), lambda i: (i + _TCG, 0)),
        ],
        out_specs=pl.BlockSpec((_TCB, _H), lambda i: (i, 0)),
        out_shape=jax.ShapeDtypeStruct((_N, _H), jnp.float32),
    )(agg, agg, xw, b, W2, hist, hist)


def _final_body(agg_ref, xw_ref, b_ref, hist_ref, batch_ref, lw_ref, lb_ref,
                o_ref):
    hA = hist_ref[: _N]
    hB = hist_ref[_N:]
    dinv = _dinv_from_hist(hA, hB)
    pre = dinv * (agg_ref[: _N] + agg_ref[_N:] + xw_ref[...]) + b_ref[...]
    h = jnp.maximum(pre, 0.0)

    batch = batch_ref[...]
    gids = lax.broadcasted_iota(jnp.int32, (_N, _G), 1)
    onehot = (batch == gids).astype(jnp.float32)
    ssum = lax.dot_general(onehot, h, (((0,), (0,)), ((), ())),
                           preferred_element_type=jnp.float32)
    cnt = lax.dot_general(onehot, jnp.ones((_N, 1), jnp.float32),
                          (((0,), (0,)), ((), ())),
                          preferred_element_type=jnp.float32)
    mean = ssum / jnp.maximum(cnt, 1.0)

    neg = jnp.float32(-jnp.inf)
    parts = []
    for g in range(_G):
        m = jnp.where(batch == g, h, neg)
        parts.append(jnp.max(m, axis=0, keepdims=True))
    mx = jnp.concatenate(parts, axis=0)

    z = jnp.concatenate([mx, mean], axis=1)
    logits = jnp.dot(z, lw_ref[...], preferred_element_type=jnp.float32) \
        + lb_ref[...]
    zmax = jnp.max(logits, axis=1, keepdims=True)
    sh = logits - zmax
    o_ref[...] = sh - jnp.log(jnp.sum(jnp.exp(sh), axis=1, keepdims=True))


def _final_tc(agg, xw, b, hist, batch2, lin_W, lin_b):
    return pl.pallas_call(
        _final_body,
        out_shape=jax.ShapeDtypeStruct((_G, _C), jnp.float32),
    )(agg, xw, b, hist, batch2, lin_W, lin_b)


def kernel(x, edge_index, batch, W1, b1, W2, b2, lin_W, lin_b):
    src2 = edge_index[0].reshape(_EROWS, _B)
    dst2 = edge_index[1].reshape(_EROWS, _B)
    batch2 = batch.reshape(_N, 1)
    b1r = b1.reshape(1, _H)
    b2r = b2.reshape(1, _H)
    lbr = lin_b.reshape(1, _C)

    hist = _deg_sc(dst2)
    xw1 = _prep_tc(x, W1, hist)
    agg1 = _agg_sc(xw1, src2, dst2)
    xw2 = _mid_tc(agg1, xw1, b1r, W2, hist)
    agg2 = _agg_sc(xw2, src2, dst2)
    return _final_tc(agg2, xw2, b2r, hist, batch2, lbr if False else lin_W, lbr)


# speedup vs baseline: 10.6336x; 10.6336x over previous
import functools

import jax
import jax.numpy as jnp
from jax import lax
from jax.experimental import pallas as pl
from jax.experimental.pallas import tpu as pltpu
from jax.experimental.pallas import tpu_sc as plsc

_N = 10000
_E = 320000
_H = 128
_G = 64
_C = 10

_B = 125
_BP = 128
_EROWS = _E // _B
_NPAD = _N + 8
_RPW = _EROWS // 32
_STG = 16
_NSTG = _RPW // _STG
_TROWS = 624
_TLAST = _N - 16 * _TROWS
_CHUNKS = (248, 248, 128)

_TCB = 1000
_TCG = _N // _TCB


def _sc_mesh():
    return plsc.VectorSubcoreMesh(core_axis_name="c", subcore_axis_name="s")


def _deg_sc(dst2, zeros16):
    @functools.partial(
        pl.kernel,
        out_type=jax.ShapeDtypeStruct((2 * _N, 16), jnp.float32),
        mesh=_sc_mesh(),
        scratch_types=[
            pltpu.VMEM((_RPW, _BP), jnp.int32),
            pltpu.VMEM((_BP, 16), jnp.float32),
            pltpu.VMEM_SHARED((_NPAD, 16), jnp.float32),
        ],
    )
    def deg_kernel(dst_hbm, zeros_hbm, out_hbm, didx, ones_v, hist):
        cid = lax.axis_index("c")
        sid = lax.axis_index("s")
        wid = cid * 16 + sid
        tid = sid

        def fill(r, _):
            ones_v[r, :] = jnp.full((16,), 1.0, jnp.float32)
            return 0

        lax.fori_loop(0, _BP, fill, 0)

        base = tid * _TROWS
        pltpu.sync_copy(zeros_hbm.at[pl.ds(base, _TROWS)],
                        hist.at[pl.ds(base, _TROWS)])

        @pl.when(tid == 15)
        def _():
            pltpu.sync_copy(zeros_hbm.at[pl.ds(16 * _TROWS, _TLAST)],
                            hist.at[pl.ds(16 * _TROWS, _TLAST)])

        plsc.subcore_barrier()

        pltpu.sync_copy(dst_hbm.at[pl.ds(wid * _RPW, _RPW)], didx)

        def body(r, _):
            pltpu.sync_copy(ones_v, hist.at[didx.at[r]], add=True)
            return 0

        lax.fori_loop(0, _RPW, body, 0)
        plsc.subcore_barrier()

        obase = cid * _N + base
        pltpu.sync_copy(hist.at[pl.ds(base, _TROWS)],
                        out_hbm.at[pl.ds(obase, _TROWS)])

        @pl.when(tid == 15)
        def _():
            pltpu.sync_copy(hist.at[pl.ds(16 * _TROWS, _TLAST)],
                            out_hbm.at[pl.ds(cid * _N + 16 * _TROWS, _TLAST)])

    return deg_kernel(dst2, zeros16)


def _agg_sc(xw, src2, dst2, zerosH):
    @functools.partial(
        pl.kernel,
        out_type=jax.ShapeDtypeStruct((2 * _N, _H), jnp.float32),
        mesh=_sc_mesh(),
        scratch_types=[
            pltpu.VMEM((_STG, _BP), jnp.int32),
            pltpu.VMEM((_STG, _BP), jnp.int32),
            pltpu.VMEM((2 * _BP, _H), jnp.float32),
            pltpu.VMEM_SHARED((_NPAD, _H), jnp.float32),
            pltpu.SemaphoreType.DMA,
        ],
    )
    def agg_kernel(xw_hbm, src_hbm, dst_hbm, zeros_hbm, out_hbm,
                   sidx, didx, buf, acc, gsem):
        cid = lax.axis_index("c")
        sid = lax.axis_index("s")
        wid = cid * 16 + sid
        tid = sid

        base = tid * _TROWS
        pltpu.sync_copy(zeros_hbm.at[pl.ds(base, _TROWS)],
                        acc.at[pl.ds(base, _TROWS)])

        @pl.when(tid == 15)
        def _():
            pltpu.sync_copy(zeros_hbm.at[pl.ds(16 * _TROWS, _TLAST)],
                            acc.at[pl.ds(16 * _TROWS, _TLAST)])

        plsc.subcore_barrier()

        ebase = wid * _RPW

        def stage(s, _):
            pltpu.sync_copy(src_hbm.at[pl.ds(ebase + s * _STG, _STG)], sidx)
            pltpu.sync_copy(dst_hbm.at[pl.ds(ebase + s * _STG, _STG)], didx)
            slot = [buf.at[pl.ds(0, _BP)], buf.at[pl.ds(_BP, _BP)]]
            for p in range(_STG // 2):
                g0 = pltpu.async_copy(
                    xw_hbm.at[sidx.at[2 * p]], slot[0], gsem)
                g1 = pltpu.async_copy(
                    xw_hbm.at[sidx.at[2 * p + 1]], slot[1], gsem)
                g0.wait()
                g1.wait()
                pltpu.sync_copy(slot[0], acc.at[didx.at[2 * p]], add=True)
                pltpu.sync_copy(slot[1], acc.at[didx.at[2 * p + 1]], add=True)
            return 0

        lax.fori_loop(0, _NSTG, stage, 0)
        plsc.subcore_barrier()

        obase = cid * _N + base
        pltpu.sync_copy(acc.at[pl.ds(base, _TROWS)],
                        out_hbm.at[pl.ds(obase, _TROWS)])

        @pl.when(tid == 15)
        def _():
            pltpu.sync_copy(acc.at[pl.ds(16 * _TROWS, _TLAST)],
                            out_hbm.at[pl.ds(cid * _N + 16 * _TROWS, _TLAST)])

    return agg_kernel(xw, src2, dst2, zerosH)


def _dinv_from_hist(hA, hB):
    deg = hA[:, 0:1] + hB[:, 0:1] + 1.0
    return lax.rsqrt(deg)


def _prep_body(x_ref, w_ref, hA_ref, hB_ref, o_ref):
    dinv = _dinv_from_hist(hA_ref[...], hB_ref[...])
    o_ref[...] = dinv * jnp.dot(x_ref[...], w_ref[...],
                                preferred_element_type=jnp.float32)


def _prep_tc(x, W, hist):
    return pl.pallas_call(
        _prep_body,
        grid=(_TCG,),
        in_specs=[
            pl.BlockSpec((_TCB, _H), lambda i: (i, 0)),
            pl.BlockSpec((_H, _H), lambda i: (0, 0)),
            pl.BlockSpec((_TCB, 16), lambda i: (i, 0)),
            pl.BlockSpec((_TCB, 16), lambda i: (i + _TCG, 0)),
        ],
        out_specs=pl.BlockSpec((_TCB, _H), lambda i: (i, 0)),
        out_shape=jax.ShapeDtypeStruct((_NPAD, _H), jnp.float32),
    )(x, W, hist, hist)


def _mid_body(a0_ref, a1_ref, xw_ref, b_ref, w_ref, hA_ref, hB_ref, o_ref):
    dinv = _dinv_from_hist(hA_ref[...], hB_ref[...])
    pre = dinv * (a0_ref[...] + a1_ref[...] + xw_ref[...]) + b_ref[...]
    h = jnp.maximum(pre, 0.0)
    o_ref[...] = dinv * jnp.dot(h, w_ref[...],
                                preferred_element_type=jnp.float32)


def _mid_tc(agg, xw, b, W2, hist):
    return pl.pallas_call(
        _mid_body,
        grid=(_TCG,),
        in_specs=[
            pl.BlockSpec((_TCB, _H), lambda i: (i, 0)),
            pl.BlockSpec((_TCB, _H), lambda i: (i + _TCG, 0)),
            pl.BlockSpec((_TCB, _H), lambda i: (i, 0)),
            pl.BlockSpec((1, _H), lambda i: (0, 0)),
            pl.BlockSpec((_H, _H), lambda i: (0, 0)),
            pl.BlockSpec((_TCB, 16), lambda i: (i, 0)),
            pl.BlockSpec((_TCB, 16), lambda i: (i + _TCG, 0)),
        ],
        out_specs=pl.BlockSpec((_TCB, _H), lambda i: (i, 0)),
        out_shape=jax.ShapeDtypeStruct((_NPAD, _H), jnp.float32),
    )(agg, agg, xw, b, W2, hist, hist)


def _final_body(agg_ref, xw_ref, b_ref, hist_ref, batch_ref, lw_ref, lb_ref,
                o_ref, mx_ref):
    hA = hist_ref[: _N]
    hB = hist_ref[_N:]
    dinv = _dinv_from_hist(hA, hB)
    pre = dinv * (agg_ref[: _N] + agg_ref[_N:] + xw_ref[: _N]) + b_ref[...]
    h = jnp.maximum(pre, 0.0)

    batch = batch_ref[...]
    gids = lax.broadcasted_iota(jnp.int32, (_N, _G), 1)
    onehot = (batch == gids).astype(jnp.float32)
    ssum = lax.dot_general(onehot, h, (((0,), (0,)), ((), ())),
                           preferred_element_type=jnp.float32)
    cnt = lax.dot_general(onehot, jnp.ones((_N, 1), jnp.float32),
                          (((0,), (0,)), ((), ())),
                          preferred_element_type=jnp.float32)
    mean = ssum / jnp.maximum(cnt, 1.0)

    neg = jnp.float32(-jnp.inf)

    def gmax(g, carry):
        m = jnp.where(batch == g, h, neg)
        mx_ref[pl.ds(g, 1), :] = jnp.max(m, axis=0, keepdims=True)
        return carry

    lax.fori_loop(0, _G, gmax, 0)
    mx = mx_ref[...]

    z = jnp.concatenate([mx, mean], axis=1)
    logits = jnp.dot(z, lw_ref[...], preferred_element_type=jnp.float32) \
        + lb_ref[...]
    zmax = jnp.max(logits, axis=1, keepdims=True)
    sh = logits - zmax
    o_ref[...] = sh - jnp.log(jnp.sum(jnp.exp(sh), axis=1, keepdims=True))


def _final_tc(agg, xw, b, hist, batch2, lin_W, lin_b):
    return pl.pallas_call(
        _final_body,
        out_shape=jax.ShapeDtypeStruct((_G, _C), jnp.float32),
        scratch_shapes=[pltpu.VMEM((_G, _H), jnp.float32)],
    )(agg, xw, b, hist, batch2, lin_W, lin_b)


def kernel(x, edge_index, batch, W1, b1, W2, b2, lin_W, lin_b):
    pad = jnp.full((_EROWS, _BP - _B), _N, jnp.int32)
    src2 = jnp.concatenate([edge_index[0].reshape(_EROWS, _B), pad], axis=1)
    dst2 = jnp.concatenate([edge_index[1].reshape(_EROWS, _B), pad], axis=1)
    batch2 = batch.reshape(_N, 1)
    b1r = b1.reshape(1, _H)
    b2r = b2.reshape(1, _H)
    lbr = lin_b.reshape(1, _C)

    zeros16 = jnp.zeros((_N, 16), jnp.float32)
    zerosH = jnp.zeros((_N, _H), jnp.float32)
    hist = _deg_sc(dst2, zeros16)
    xw1 = _prep_tc(x, W1, hist)
    agg1 = _agg_sc(xw1, src2, dst2, zerosH)
    xw2 = _mid_tc(agg1, xw1, b1r, W2, hist)
    agg2 = _agg_sc(xw2, src2, dst2, zerosH)
    return _final_tc(agg2, xw2, b2r, hist, batch2, lin_W, lbr)


# speedup vs baseline: 10.6582x; 1.0023x over previous
import functools

import jax
import jax.numpy as jnp
from jax import lax
from jax.experimental import pallas as pl
from jax.experimental.pallas import tpu as pltpu
from jax.experimental.pallas import tpu_sc as plsc

_N = 10000
_E = 320000
_H = 128
_G = 64
_C = 10

_B = 125
_BP = 128
_EROWS = _E // _B
_NPAD = _N + 8
_RPW = _EROWS // 32
_STG = 16
_NSTG = _RPW // _STG
_TROWS = 624
_TLAST = _N - 16 * _TROWS
_CHUNKS = (248, 248, 128)

_TCB = 1000
_TCG = _N // _TCB


def _sc_mesh():
    return plsc.VectorSubcoreMesh(core_axis_name="c", subcore_axis_name="s")


def _deg_sc(dst2, zeros16):
    @functools.partial(
        pl.kernel,
        out_type=jax.ShapeDtypeStruct((2 * _N, 16), jnp.float32),
        mesh=_sc_mesh(),
        scratch_types=[
            pltpu.VMEM((_RPW, _BP), jnp.int32),
            pltpu.VMEM((_BP, 16), jnp.float32),
            pltpu.VMEM_SHARED((_NPAD, 16), jnp.float32),
        ],
    )
    def deg_kernel(dst_hbm, zeros_hbm, out_hbm, didx, ones_v, hist):
        cid = lax.axis_index("c")
        sid = lax.axis_index("s")
        wid = cid * 16 + sid
        tid = sid

        def fill(r, _):
            ones_v[r, :] = jnp.full((16,), 1.0, jnp.float32)
            return 0

        lax.fori_loop(0, _BP, fill, 0)

        base = tid * _TROWS
        pltpu.sync_copy(zeros_hbm.at[pl.ds(base, _TROWS)],
                        hist.at[pl.ds(base, _TROWS)])

        @pl.when(tid == 15)
        def _():
            pltpu.sync_copy(zeros_hbm.at[pl.ds(16 * _TROWS, _TLAST)],
                            hist.at[pl.ds(16 * _TROWS, _TLAST)])

        plsc.subcore_barrier()

        pltpu.sync_copy(dst_hbm.at[pl.ds(wid * _RPW, _RPW)], didx)

        def body(r, _):
            pltpu.sync_copy(ones_v, hist.at[didx.at[r]], add=True)
            return 0

        lax.fori_loop(0, _RPW, body, 0)
        plsc.subcore_barrier()

        obase = cid * _N + base
        pltpu.sync_copy(hist.at[pl.ds(base, _TROWS)],
                        out_hbm.at[pl.ds(obase, _TROWS)])

        @pl.when(tid == 15)
        def _():
            pltpu.sync_copy(hist.at[pl.ds(16 * _TROWS, _TLAST)],
                            out_hbm.at[pl.ds(cid * _N + 16 * _TROWS, _TLAST)])

    return deg_kernel(dst2, zeros16)


def _agg_sc(xw, src2, dst2, zerosH):
    @functools.partial(
        pl.kernel,
        out_type=jax.ShapeDtypeStruct((2 * _N, _H), jnp.float32),
        mesh=_sc_mesh(),
        scratch_types=[
            pltpu.VMEM((_STG, _BP), jnp.int32),
            pltpu.VMEM((_STG, _BP), jnp.int32),
            pltpu.VMEM((2 * _BP, _H), jnp.float32),
            pltpu.VMEM_SHARED((_NPAD, _H), jnp.float32),
            pltpu.SemaphoreType.DMA,
            pltpu.SemaphoreType.DMA,
            pltpu.SemaphoreType.DMA,
            pltpu.SemaphoreType.DMA,
        ],
    )
    def agg_kernel(xw_hbm, src_hbm, dst_hbm, zeros_hbm, out_hbm,
                   sidx, didx, buf, acc, gsem0, gsem1, ssem0, ssem1):
        cid = lax.axis_index("c")
        sid = lax.axis_index("s")
        wid = cid * 16 + sid
        tid = sid

        base = tid * _TROWS
        pltpu.sync_copy(zeros_hbm.at[pl.ds(base, _TROWS)],
                        acc.at[pl.ds(base, _TROWS)])

        @pl.when(tid == 15)
        def _():
            pltpu.sync_copy(zeros_hbm.at[pl.ds(16 * _TROWS, _TLAST)],
                            acc.at[pl.ds(16 * _TROWS, _TLAST)])

        plsc.subcore_barrier()

        ebase = wid * _RPW

        def stage(s, _):
            pltpu.sync_copy(src_hbm.at[pl.ds(ebase + s * _STG, _STG)], sidx)
            pltpu.sync_copy(dst_hbm.at[pl.ds(ebase + s * _STG, _STG)], didx)
            slot = [buf.at[pl.ds(0, _BP)], buf.at[pl.ds(_BP, _BP)]]
            for p in range(_STG // 2):
                g0 = pltpu.async_copy(
                    xw_hbm.at[sidx.at[2 * p]], slot[0], gsem0)
                g1 = pltpu.async_copy(
                    xw_hbm.at[sidx.at[2 * p + 1]], slot[1], gsem1)
                g0.wait()
                g1.wait()
                pltpu.sync_copy(slot[0], acc.at[didx.at[2 * p]], add=True)
                pltpu.sync_copy(slot[1], acc.at[didx.at[2 * p + 1]], add=True)
            return 0

        lax.fori_loop(0, _NSTG, stage, 0)
        plsc.subcore_barrier()

        obase = cid * _N + base
        pltpu.sync_copy(acc.at[pl.ds(base, _TROWS)],
                        out_hbm.at[pl.ds(obase, _TROWS)])

        @pl.when(tid == 15)
        def _():
            pltpu.sync_copy(acc.at[pl.ds(16 * _TROWS, _TLAST)],
                            out_hbm.at[pl.ds(cid * _N + 16 * _TROWS, _TLAST)])

    return agg_kernel(xw, src2, dst2, zerosH)


def _dinv_from_hist(hA, hB):
    deg = hA[:, 0:1] + hB[:, 0:1] + 1.0
    return lax.rsqrt(deg)


def _prep_body(x_ref, w_ref, hA_ref, hB_ref, o_ref):
    dinv = _dinv_from_hist(hA_ref[...], hB_ref[...])
    o_ref[...] = dinv * jnp.dot(x_ref[...], w_ref[...],
                                preferred_element_type=jnp.float32)


def _prep_tc(x, W, hist):
    return pl.pallas_call(
        _prep_body,
        grid=(_TCG,),
        in_specs=[
            pl.BlockSpec((_TCB, _H), lambda i: (i, 0)),
            pl.BlockSpec((_H, _H), lambda i: (0, 0)),
            pl.BlockSpec((_TCB, 16), lambda i: (i, 0)),
            pl.BlockSpec((_TCB, 16), lambda i: (i + _TCG, 0)),
        ],
        out_specs=pl.BlockSpec((_TCB, _H), lambda i: (i, 0)),
        out_shape=jax.ShapeDtypeStruct((_NPAD, _H), jnp.float32),
    )(x, W, hist, hist)


def _mid_body(a0_ref, a1_ref, xw_ref, b_ref, w_ref, hA_ref, hB_ref, o_ref):
    dinv = _dinv_from_hist(hA_ref[...], hB_ref[...])
    pre = dinv * (a0_ref[...] + a1_ref[...] + xw_ref[...]) + b_ref[...]
    h = jnp.maximum(pre, 0.0)
    o_ref[...] = dinv * jnp.dot(h, w_ref[...],
                                preferred_element_type=jnp.float32)


def _mid_tc(agg, xw, b, W2, hist):
    return pl.pallas_call(
        _mid_body,
        grid=(_TCG,),
        in_specs=[
            pl.BlockSpec((_TCB, _H), lambda i: (i, 0)),
            pl.BlockSpec((_TCB, _H), lambda i: (i + _TCG, 0)),
            pl.BlockSpec((_TCB, _H), lambda i: (i, 0)),
            pl.BlockSpec((1, _H), lambda i: (0, 0)),
            pl.BlockSpec((_H, _H), lambda i: (0, 0)),
            pl.BlockSpec((_TCB, 16), lambda i: (i, 0)),
            pl.BlockSpec((_TCB, 16), lambda i: (i + _TCG, 0)),
        ],
        out_specs=pl.BlockSpec((_TCB, _H), lambda i: (i, 0)),
        out_shape=jax.ShapeDtypeStruct((_NPAD, _H), jnp.float32),
    )(agg, agg, xw, b, W2, hist, hist)


def _final_body(agg_ref, xw_ref, b_ref, hist_ref, batch_ref, lw_ref, lb_ref,
                o_ref, mx_ref):
    hA = hist_ref[: _N]
    hB = hist_ref[_N:]
    dinv = _dinv_from_hist(hA, hB)
    pre = dinv * (agg_ref[: _N] + agg_ref[_N:] + xw_ref[: _N]) + b_ref[...]
    h = jnp.maximum(pre, 0.0)

    batch = batch_ref[...]
    gids = lax.broadcasted_iota(jnp.int32, (_N, _G), 1)
    onehot = (batch == gids).astype(jnp.float32)
    ssum = lax.dot_general(onehot, h, (((0,), (0,)), ((), ())),
                           preferred_element_type=jnp.float32)
    cnt = lax.dot_general(onehot, jnp.ones((_N, 1), jnp.float32),
                          (((0,), (0,)), ((), ())),
                          preferred_element_type=jnp.float32)
    mean = ssum / jnp.maximum(cnt, 1.0)

    neg = jnp.float32(-jnp.inf)

    def gmax(g, carry):
        m = jnp.where(batch == g, h, neg)
        mx_ref[pl.ds(g, 1), :] = jnp.max(m, axis=0, keepdims=True)
        return carry

    lax.fori_loop(0, _G, gmax, 0)
    mx = mx_ref[...]

    z = jnp.concatenate([mx, mean], axis=1)
    logits = jnp.dot(z, lw_ref[...], preferred_element_type=jnp.float32) \
        + lb_ref[...]
    zmax = jnp.max(logits, axis=1, keepdims=True)
    sh = logits - zmax
    o_ref[...] = sh - jnp.log(jnp.sum(jnp.exp(sh), axis=1, keepdims=True))


def _final_tc(agg, xw, b, hist, batch2, lin_W, lin_b):
    return pl.pallas_call(
        _final_body,
        out_shape=jax.ShapeDtypeStruct((_G, _C), jnp.float32),
        scratch_shapes=[pltpu.VMEM((_G, _H), jnp.float32)],
    )(agg, xw, b, hist, batch2, lin_W, lin_b)


def kernel(x, edge_index, batch, W1, b1, W2, b2, lin_W, lin_b):
    pad = jnp.full((_EROWS, _BP - _B), _N, jnp.int32)
    src2 = jnp.concatenate([edge_index[0].reshape(_EROWS, _B), pad], axis=1)
    dst2 = jnp.concatenate([edge_index[1].reshape(_EROWS, _B), pad], axis=1)
    batch2 = batch.reshape(_N, 1)
    b1r = b1.reshape(1, _H)
    b2r = b2.reshape(1, _H)
    lbr = lin_b.reshape(1, _C)

    zeros16 = jnp.zeros((_N, 16), jnp.float32)
    zerosH = jnp.zeros((_N, _H), jnp.float32)
    hist = _deg_sc(dst2, zeros16)
    xw1 = _prep_tc(x, W1, hist)
    agg1 = _agg_sc(xw1, src2, dst2, zerosH)
    xw2 = _mid_tc(agg1, xw1, b1r, W2, hist)
    agg2 = _agg_sc(xw2, src2, dst2, zerosH)
    return _final_tc(agg2, xw2, b2r, hist, batch2, lin_W, lbr)


# speedup vs baseline: 10.6612x; 1.0003x over previous
import functools

import jax
import jax.numpy as jnp
from jax import lax
from jax.experimental import pallas as pl
from jax.experimental.pallas import tpu as pltpu
from jax.experimental.pallas import tpu_sc as plsc

_N = 10000
_E = 320000
_H = 128
_G = 64
_C = 10

_B = 125
_BP = 128
_EROWS = _E // _B
_NPAD = _N + 8
_RPW = _EROWS // 32
_STG = 16
_NSTG = _RPW // _STG
_TROWS = 624
_TLAST = _N - 16 * _TROWS

_TCB = 1000
_TCG = _N // _TCB


def _sc_mesh():
    return plsc.VectorSubcoreMesh(core_axis_name="c", subcore_axis_name="s")


def _deg_sc(dst2, zeros16):
    @functools.partial(
        pl.kernel,
        out_type=jax.ShapeDtypeStruct((2 * _N, 16), jnp.float32),
        mesh=_sc_mesh(),
        scratch_types=[
            pltpu.VMEM((_RPW, _BP), jnp.int32),
            pltpu.VMEM((_BP, 16), jnp.float32),
            pltpu.VMEM_SHARED((_NPAD, 16), jnp.float32),
        ],
    )
    def deg_kernel(dst_hbm, zeros_hbm, out_hbm, didx, ones_v, hist):
        cid = lax.axis_index("c")
        sid = lax.axis_index("s")
        wid = cid * 16 + sid
        tid = sid

        def fill(r, _):
            ones_v[r, :] = jnp.full((16,), 1.0, jnp.float32)
            return 0

        lax.fori_loop(0, _BP, fill, 0)

        base = tid * _TROWS
        pltpu.sync_copy(zeros_hbm.at[pl.ds(base, _TROWS)],
                        hist.at[pl.ds(base, _TROWS)])

        @pl.when(tid == 15)
        def _():
            pltpu.sync_copy(zeros_hbm.at[pl.ds(16 * _TROWS, _TLAST)],
                            hist.at[pl.ds(16 * _TROWS, _TLAST)])

        plsc.subcore_barrier()

        pltpu.sync_copy(dst_hbm.at[pl.ds(wid * _RPW, _RPW)], didx)

        def body(r, _):
            pltpu.sync_copy(ones_v, hist.at[didx.at[r]], add=True)
            return 0

        lax.fori_loop(0, _RPW, body, 0)
        plsc.subcore_barrier()

        obase = cid * _N + base
        pltpu.sync_copy(hist.at[pl.ds(base, _TROWS)],
                        out_hbm.at[pl.ds(obase, _TROWS)])

        @pl.when(tid == 15)
        def _():
            pltpu.sync_copy(hist.at[pl.ds(16 * _TROWS, _TLAST)],
                            out_hbm.at[pl.ds(cid * _N + 16 * _TROWS, _TLAST)])

    return deg_kernel(dst2, zeros16)


def _agg_sc(xw, src2, dst2, zerosH):
    @functools.partial(
        pl.kernel,
        out_type=jax.ShapeDtypeStruct((2 * _N, _H), jnp.float32),
        mesh=_sc_mesh(),
        scratch_types=[
            pltpu.VMEM((_STG, _BP), jnp.int32),
            pltpu.VMEM((_STG, _BP), jnp.int32),
            pltpu.VMEM((2 * _BP, _H), jnp.float32),
            pltpu.VMEM_SHARED((_NPAD, _H), jnp.float32),
            pltpu.SemaphoreType.DMA,
            pltpu.SemaphoreType.DMA,
        ],
    )
    def agg_kernel(xw_hbm, src_hbm, dst_hbm, zeros_hbm, out_hbm,
                   sidx, didx, buf, acc, gsem0, gsem1):
        cid = lax.axis_index("c")
        sid = lax.axis_index("s")
        wid = cid * 16 + sid
        tid = sid

        base = tid * _TROWS
        pltpu.sync_copy(zeros_hbm.at[pl.ds(base, _TROWS)],
                        acc.at[pl.ds(base, _TROWS)])

        @pl.when(tid == 15)
        def _():
            pltpu.sync_copy(zeros_hbm.at[pl.ds(16 * _TROWS, _TLAST)],
                            acc.at[pl.ds(16 * _TROWS, _TLAST)])

        plsc.subcore_barrier()

        ebase = wid * _RPW

        def stage(s, _):
            pltpu.sync_copy(src_hbm.at[pl.ds(ebase + s * _STG, _STG)], sidx)
            pltpu.sync_copy(dst_hbm.at[pl.ds(ebase + s * _STG, _STG)], didx)
            slot = [buf.at[pl.ds(0, _BP)], buf.at[pl.ds(_BP, _BP)]]
            for p in range(_STG // 2):
                g0 = pltpu.async_copy(
                    xw_hbm.at[sidx.at[2 * p]], slot[0], gsem0)
                g1 = pltpu.async_copy(
                    xw_hbm.at[sidx.at[2 * p + 1]], slot[1], gsem1)
                g0.wait()
                g1.wait()
                pltpu.sync_copy(slot[0], acc.at[didx.at[2 * p]], add=True)
                pltpu.sync_copy(slot[1], acc.at[didx.at[2 * p + 1]], add=True)
            return 0

        lax.fori_loop(0, _NSTG, stage, 0)
        plsc.subcore_barrier()

        obase = cid * _N + base
        pltpu.sync_copy(acc.at[pl.ds(base, _TROWS)],
                        out_hbm.at[pl.ds(obase, _TROWS)])

        @pl.when(tid == 15)
        def _():
            pltpu.sync_copy(acc.at[pl.ds(16 * _TROWS, _TLAST)],
                            out_hbm.at[pl.ds(cid * _N + 16 * _TROWS, _TLAST)])

    return agg_kernel(xw, src2, dst2, zerosH)


def _dinv_from_hist(hA, hB):
    deg = hA[:, 0:1] + hB[:, 0:1] + 1.0
    return lax.rsqrt(deg)


def _prep_body(x_ref, w_ref, hA_ref, hB_ref, o_ref):
    dinv = _dinv_from_hist(hA_ref[...], hB_ref[...])
    o_ref[...] = dinv * jnp.dot(x_ref[...], w_ref[...],
                                preferred_element_type=jnp.float32)


def _prep_tc(x, W, hist):
    return pl.pallas_call(
        _prep_body,
        grid=(_TCG,),
        in_specs=[
            pl.BlockSpec((_TCB, _H), lambda i: (i, 0)),
            pl.BlockSpec((_H, _H), lambda i: (0, 0)),
            pl.BlockSpec((_TCB, 16), lambda i: (i, 0)),
            pl.BlockSpec((_TCB, 16), lambda i: (i + _TCG, 0)),
        ],
        out_specs=pl.BlockSpec((_TCB, _H), lambda i: (i, 0)),
        out_shape=jax.ShapeDtypeStruct((_NPAD, _H), jnp.float32),
    )(x, W, hist, hist)


def _mid_body(a0_ref, a1_ref, xw_ref, b_ref, w_ref, hA_ref, hB_ref, o_ref):
    dinv = _dinv_from_hist(hA_ref[...], hB_ref[...])
    pre = dinv * (a0_ref[...] + a1_ref[...] + xw_ref[...]) + b_ref[...]
    h = jnp.maximum(pre, 0.0)
    o_ref[...] = dinv * jnp.dot(h, w_ref[...],
                                preferred_element_type=jnp.float32)


def _mid_tc(agg, xw, b, W2, hist):
    return pl.pallas_call(
        _mid_body,
        grid=(_TCG,),
        in_specs=[
            pl.BlockSpec((_TCB, _H), lambda i: (i, 0)),
            pl.BlockSpec((_TCB, _H), lambda i: (i + _TCG, 0)),
            pl.BlockSpec((_TCB, _H), lambda i: (i, 0)),
            pl.BlockSpec((1, _H), lambda i: (0, 0)),
            pl.BlockSpec((_H, _H), lambda i: (0, 0)),
            pl.BlockSpec((_TCB, 16), lambda i: (i, 0)),
            pl.BlockSpec((_TCB, 16), lambda i: (i + _TCG, 0)),
        ],
        out_specs=pl.BlockSpec((_TCB, _H), lambda i: (i, 0)),
        out_shape=jax.ShapeDtypeStruct((_NPAD, _H), jnp.float32),
    )(agg, agg, xw, b, W2, hist, hist)


def _final_body(agg_ref, xw_ref, b_ref, hist_ref, batch_ref, lw_ref, lb_ref,
                o_ref, mx_ref):
    hA = hist_ref[: _N]
    hB = hist_ref[_N:]
    dinv = _dinv_from_hist(hA, hB)
    pre = dinv * (agg_ref[: _N] + agg_ref[_N:] + xw_ref[: _N]) + b_ref[...]
    h = jnp.maximum(pre, 0.0)

    batch = batch_ref[...]
    gids = lax.broadcasted_iota(jnp.int32, (_N, _G), 1)
    onehot = (batch == gids).astype(jnp.float32)
    ssum = lax.dot_general(onehot, h, (((0,), (0,)), ((), ())),
                           preferred_element_type=jnp.float32)
    cnt = lax.dot_general(onehot, jnp.ones((_N, 1), jnp.float32),
                          (((0,), (0,)), ((), ())),
                          preferred_element_type=jnp.float32)
    mean = ssum / jnp.maximum(cnt, 1.0)

    neg = jnp.float32(-jnp.inf)

    def gmax(g, carry):
        m = jnp.where(batch == g, h, neg)
        mx_ref[pl.ds(g, 1), :] = jnp.max(m, axis=0, keepdims=True)
        return carry

    lax.fori_loop(0, _G, gmax, 0)
    mx = mx_ref[...]

    z = jnp.concatenate([mx, mean], axis=1)
    logits = jnp.dot(z, lw_ref[...], preferred_element_type=jnp.float32) \
        + lb_ref[...]
    zmax = jnp.max(logits, axis=1, keepdims=True)
    sh = logits - zmax
    o_ref[...] = sh - jnp.log(jnp.sum(jnp.exp(sh), axis=1, keepdims=True))


def _final_tc(agg, xw, b, hist, batch2, lin_W, lin_b):
    return pl.pallas_call(
        _final_body,
        out_shape=jax.ShapeDtypeStruct((_G, _C), jnp.float32),
        scratch_shapes=[pltpu.VMEM((_G, _H), jnp.float32)],
    )(agg, xw, b, hist, batch2, lin_W, lin_b)


def kernel(x, edge_index, batch, W1, b1, W2, b2, lin_W, lin_b):
    pad = jnp.full((_EROWS, _BP - _B), _N, jnp.int32)
    src2 = jnp.concatenate([edge_index[0].reshape(_EROWS, _B), pad], axis=1)
    dst2 = jnp.concatenate([edge_index[1].reshape(_EROWS, _B), pad], axis=1)
    batch2 = batch.reshape(_N, 1)
    b1r = b1.reshape(1, _H)
    b2r = b2.reshape(1, _H)
    lbr = lin_b.reshape(1, _C)

    zeros16 = jnp.zeros((_N, 16), jnp.float32)
    zerosH = jnp.zeros((_N, _H), jnp.float32)
    hist = _deg_sc(dst2, zeros16)
    xw1 = _prep_tc(x, W1, hist)
    agg1 = _agg_sc(xw1, src2, dst2, zerosH)
    xw2 = _mid_tc(agg1, xw1, b1r, W2, hist)
    agg2 = _agg_sc(xw2, src2, dst2, zerosH)
    return _final_tc(agg2, xw2, b2r, hist, batch2, lin_W, lbr)
